# Initial kernel scaffold; baseline (speedup 1.0000x reference)
import jax
import jax.numpy as jnp
from jax.experimental import pallas as pl


def kernel(z_pos, z_features, graph_pos, z_batch, graph_batch, sage_l_w, sage_l_b, sage_r_w, sage_r_b, w0, b0, w1, b1, w2, b2):
    raise NotImplementedError("write your pallas kernel here")


# speedup vs baseline: 8.5081x; 8.5081x over previous
import functools

import jax
import jax.numpy as jnp
import numpy as np
from jax import lax
from jax.experimental import pallas as pl
from jax.experimental.pallas import tpu as pltpu
from jax.experimental.pallas import tpu_sc as plsc

NZ = 4096
NG = 20000
NGP = 20480
LD = 16
NMOD = 512
KG = 8
KI = 3
RB1 = 256
NB1 = NZ // RB1
RB2 = 256
NB2 = NGP // RB2
RB4 = 512
NB4 = NGP // RB4

SC_WORKERS = 32
PW = NGP // SC_WORKERS
CH = 16
NCH = PW // CH
RPC = CH * KI

_BIGI = np.int32(2 ** 30)
_INF = np.float32(np.inf)


def _feats_body(zp_ref, zt_ref, zf_ref, lw_ref, rw_ref, lb_ref, rb_ref,
                out_ref, d2_ref, acc_ref):
    b = pl.program_id(0)
    xcol = zp_ref[:, 0:1]
    ycol = zp_ref[:, 1:2]
    zx = zt_ref[0:1, :]
    zy = zt_ref[1:2, :]
    dx = xcol - zx
    dy = ycol - zy
    d2 = dx * dx + dy * dy
    rows = lax.broadcasted_iota(jnp.int32, (RB1, NZ), 0) + b * RB1
    cols = lax.broadcasted_iota(jnp.int32, (RB1, NZ), 1)
    d2_ref[...] = jnp.where(rows == cols, _INF, d2)
    acc_ref[...] = jnp.zeros((RB1, NZ), jnp.float32)
    for _ in range(KG):
        d2c = d2_ref[...]
        m = jnp.min(d2c, axis=1, keepdims=True)
        sel = jnp.min(jnp.where(d2c == m, cols, _BIGI), axis=1, keepdims=True)
        onehot = cols == sel
        acc_ref[...] = acc_ref[...] + onehot.astype(jnp.float32)
        d2_ref[...] = jnp.where(onehot, _INF, d2c)
    agg = lax.dot_general(acc_ref[...], zf_ref[...], (((1,), (0,)), ((), ())),
                          preferred_element_type=jnp.float32) * 0.125
    zfb = zf_ref[pl.ds(b * RB1, RB1), :]
    f = (lax.dot_general(agg, lw_ref[...], (((1,), (1,)), ((), ())),
                         preferred_element_type=jnp.float32)
         + lb_ref[...]
         + lax.dot_general(zfb, rw_ref[...], (((1,), (1,)), ((), ())),
                           preferred_element_type=jnp.float32)
         + rb_ref[...])
    out_ref[...] = f


def _knn_body(gp_ref, zt_ref, idx_ref, wts_ref, d2_ref):
    xcol = gp_ref[:, 0:1]
    ycol = gp_ref[:, 1:2]
    zx = zt_ref[0:1, :]
    zy = zt_ref[1:2, :]
    dx = xcol - zx
    dy = ycol - zy
    d2_ref[...] = dx * dx + dy * dy
    cols = lax.broadcasted_iota(jnp.int32, (RB2, NZ), 1)
    sels = []
    ws = []
    for _ in range(KI):
        d2c = d2_ref[...]
        m = jnp.min(d2c, axis=1, keepdims=True)
        sel = jnp.min(jnp.where(d2c == m, cols, _BIGI), axis=1, keepdims=True)
        onehot = cols == sel
        d2_ref[...] = jnp.where(onehot, _INF, d2c)
        sels.append(sel)
        ws.append(1.0 / jnp.maximum(m, 1e-16))
    wsum = ws[0] + ws[1] + ws[2]
    idx_ref[0] = jnp.concatenate(sels + sels + [sels[0], sels[1]], axis=1)
    wts_ref[0] = jnp.concatenate(ws + [wsum, wsum, wsum, wsum, wsum], axis=1)


def _mlp_body(gp_ref, fi_ref, bands_ref, w0_ref, b0_ref, w1_ref, b1_ref,
              w2_ref, b2_ref, out_ref):
    x = gp_ref[:, 0:1]
    y = gp_ref[:, 1:2]
    bands = bands_ref[...]
    xw = x * bands
    yw = y * bands
    encp = jnp.concatenate(
        [jnp.sin(xw), jnp.sin(yw), jnp.cos(xw), jnp.cos(yw)], axis=1)
    h = (lax.dot_general(encp, w0_ref[...], (((1,), (1,)), ((), ())),
                         preferred_element_type=jnp.float32)
         + b0_ref[...] + fi_ref[:, 0:256])
    h = jnp.maximum(h, 0.0)
    h = (lax.dot_general(h, w1_ref[...], (((1,), (1,)), ((), ())),
                         preferred_element_type=jnp.float32)
         + b1_ref[...] + fi_ref[:, 256:512])
    h = jnp.maximum(h, 0.0)
    o = jnp.sum(h * w2_ref[...], axis=1, keepdims=True) + b2_ref[0]
    out_ref[...] = o


def _sc_gather_body(table_hbm, idx_hbm, w_hbm, out_hbm,
                    idx_v, w_v, rows_v, out_v, sem):
    wid = lax.axis_index("s") * 2 + lax.axis_index("c")
    base_pt = wid * PW
    pltpu.sync_copy(idx_hbm.at[pl.ds(wid * PW * KI, PW * KI)], idx_v)
    pltpu.sync_copy(w_hbm.at[pl.ds(wid * PW * 4, PW * 4)], w_v)

    def chunk_body(c, carry):
        pltpu.async_copy(table_hbm.at[idx_v.at[pl.ds(c * RPC, RPC)]],
                         rows_v, sem).wait()

        def pt_body(p, carry2):
            gp = c * CH + p
            w0v = plsc.load_gather(w_v, [jnp.full((16,), 4 * gp + 0, jnp.int32)])
            w1v = plsc.load_gather(w_v, [jnp.full((16,), 4 * gp + 1, jnp.int32)])
            w2v = plsc.load_gather(w_v, [jnp.full((16,), 4 * gp + 2, jnp.int32)])
            wsv = plsc.load_gather(w_v, [jnp.full((16,), 4 * gp + 3, jnp.int32)])
            for j in range(NMOD // 16):
                sl = pl.ds(16 * j, 16)
                acc = (rows_v[KI * p, sl] * w0v
                       + rows_v[KI * p + 1, sl] * w1v
                       + rows_v[KI * p + 2, sl] * w2v)
                out_v[p, sl] = acc / wsv
            return carry2

        lax.fori_loop(0, CH, pt_body, 0)
        pltpu.sync_copy(out_v, out_hbm.at[pl.ds(base_pt + c * CH, CH)])
        return carry

    lax.fori_loop(0, NCH, chunk_body, 0)


def _make_feats_call():
    return pl.pallas_call(
        _feats_body,
        grid=(NB1,),
        in_specs=[
            pl.BlockSpec((RB1, 2), lambda b: (b, 0)),
            pl.BlockSpec((2, NZ), lambda b: (0, 0)),
            pl.BlockSpec((NZ, LD), lambda b: (0, 0)),
            pl.BlockSpec((NMOD, LD), lambda b: (0, 0)),
            pl.BlockSpec((NMOD, LD), lambda b: (0, 0)),
            pl.BlockSpec((1, NMOD), lambda b: (0, 0)),
            pl.BlockSpec((1, NMOD), lambda b: (0, 0)),
        ],
        out_specs=pl.BlockSpec((RB1, NMOD), lambda b: (b, 0)),
        out_shape=jax.ShapeDtypeStruct((NZ, NMOD), jnp.float32),
        scratch_shapes=[
            pltpu.VMEM((RB1, NZ), jnp.float32),
            pltpu.VMEM((RB1, NZ), jnp.float32),
        ],
    )


def _make_knn_call():
    return pl.pallas_call(
        _knn_body,
        grid=(NB2,),
        in_specs=[
            pl.BlockSpec((RB2, 2), lambda b: (b, 0)),
            pl.BlockSpec((2, NZ), lambda b: (0, 0)),
        ],
        out_specs=[
            pl.BlockSpec((1, RB2, 8), lambda b: (b, 0, 0)),
            pl.BlockSpec((1, RB2, 8), lambda b: (b, 0, 0)),
        ],
        out_shape=[
            jax.ShapeDtypeStruct((NB2, RB2, 8), jnp.int32),
            jax.ShapeDtypeStruct((NB2, RB2, 8), jnp.float32),
        ],
        scratch_shapes=[pltpu.VMEM((RB2, NZ), jnp.float32)],
    )


def _make_mlp_call():
    return pl.pallas_call(
        _mlp_body,
        grid=(NB4,),
        in_specs=[
            pl.BlockSpec((RB4, 2), lambda b: (b, 0)),
            pl.BlockSpec((RB4, NMOD), lambda b: (b, 0)),
            pl.BlockSpec((1, 32), lambda b: (0, 0)),
            pl.BlockSpec((256, 128), lambda b: (0, 0)),
            pl.BlockSpec((1, 256), lambda b: (0, 0)),
            pl.BlockSpec((256, 256), lambda b: (0, 0)),
            pl.BlockSpec((1, 256), lambda b: (0, 0)),
            pl.BlockSpec((1, 256), lambda b: (0, 0)),
            pl.BlockSpec(memory_space=pltpu.SMEM),
        ],
        out_specs=pl.BlockSpec((RB4, 1), lambda b: (b, 0)),
        out_shape=jax.ShapeDtypeStruct((NGP, 1), jnp.float32),
    )


def _make_sc_gather():
    mesh = plsc.VectorSubcoreMesh(core_axis_name="c", subcore_axis_name="s")
    return functools.partial(
        pl.kernel,
        mesh=mesh,
        out_type=jax.ShapeDtypeStruct((NGP, NMOD), jnp.float32),
        scratch_types=[
            pltpu.VMEM((PW * KI,), jnp.int32),
            pltpu.VMEM((PW * 4,), jnp.float32),
            pltpu.VMEM((RPC, NMOD), jnp.float32),
            pltpu.VMEM((CH, NMOD), jnp.float32),
            pltpu.SemaphoreType.DMA,
        ],
        compiler_params=pltpu.CompilerParams(needs_layout_passes=False),
    )(_sc_gather_body)


_PERM = np.empty((128,), np.int32)
_f = np.arange(32)
_PERM[_f] = 2 * _f
_PERM[32 + _f] = 2 * _f + 1
_PERM[64 + _f] = 64 + 2 * _f
_PERM[96 + _f] = 64 + 2 * _f + 1


def kernel(z_pos, z_features, graph_pos, z_batch, graph_batch,
           sage_l_w, sage_l_b, sage_r_w, sage_r_b,
           w0, b0, w1, b1, w2, b2):
    f32 = jnp.float32
    zt = z_pos.T

    feats = _make_feats_call()(
        z_pos, zt, z_features, sage_l_w, sage_r_w,
        sage_l_b[None, :], sage_r_b[None, :])

    gpad = jnp.concatenate(
        [graph_pos, jnp.zeros((NGP - NG, 2), f32)], axis=0)
    idx3, wts = _make_knn_call()(gpad, zt)
    idx_flat = idx3[:, :, :KI].reshape(-1)
    w_flat = wts[:, :, :4].reshape(-1)

    fi = _make_sc_gather()(feats, idx_flat, w_flat)

    bands = (2.0 ** jnp.linspace(0.0, 10.0, 32)).astype(f32)[None, :]
    w0p = w0[:, _PERM]
    out = _make_mlp_call()(
        gpad, fi, bands, w0p, b0[None, :], w1, b1[None, :], w2, b2)
    return out[:NG]


# speedup vs baseline: 10.6684x; 1.2539x over previous
import functools

import jax
import jax.numpy as jnp
import numpy as np
from jax import lax
from jax.experimental import pallas as pl
from jax.experimental.pallas import tpu as pltpu
from jax.experimental.pallas import tpu_sc as plsc

NZ = 4096
NG = 20000
NGP = 20480
LD = 16
NMOD = 512
KG = 8
KI = 3
RB1 = 256
NB1 = NZ // RB1
RB2 = 256
NB2 = NGP // RB2
RB4 = 512
NB4 = NGP // RB4

SC_WORKERS = 32
PW = NGP // SC_WORKERS
CH = 16
NCH = PW // CH
RPC = CH * KI

_BIGF = np.float32(1e9)
_INF = np.float32(np.inf)


def _feats_body(zp_ref, zt_ref, zf_ref, lw_ref, rw_ref, lb_ref, rb_ref,
                out_ref):
    b = pl.program_id(0)
    xcol = zp_ref[:, 0:1]
    ycol = zp_ref[:, 1:2]
    zx = zt_ref[0:1, :]
    zy = zt_ref[1:2, :]
    dx = xcol - zx
    dy = ycol - zy
    d2 = dx * dx + dy * dy
    rowsf = (lax.broadcasted_iota(jnp.int32, (RB1, NZ), 0).astype(jnp.float32)
             + (b * RB1).astype(jnp.float32))
    colsf = lax.broadcasted_iota(jnp.int32, (RB1, NZ), 1).astype(jnp.float32)
    diag = rowsf == colsf
    d2 = jnp.where(diag, _INF, d2)
    m = jnp.min(d2, axis=1, keepdims=True)
    for k in range(KG):
        eq = d2 == m
        sel = jnp.min(jnp.where(eq, colsf, _BIGF), axis=1, keepdims=True)
        d2 = jnp.where(colsf == sel, _INF, d2)
        if k < KG - 1:
            m = jnp.min(d2, axis=1, keepdims=True)
    wsel = jnp.where(jnp.isinf(d2) & (~diag), 1.0, 0.0).astype(jnp.float32)
    agg = lax.dot_general(wsel, zf_ref[...], (((1,), (0,)), ((), ())),
                          preferred_element_type=jnp.float32) * 0.125
    zfb = zf_ref[pl.ds(b * RB1, RB1), :]
    f = (lax.dot_general(agg, lw_ref[...], (((1,), (1,)), ((), ())),
                         preferred_element_type=jnp.float32)
         + lb_ref[...]
         + lax.dot_general(zfb, rw_ref[...], (((1,), (1,)), ((), ())),
                           preferred_element_type=jnp.float32)
         + rb_ref[...])
    out_ref[...] = f


def _knn_body(gp_ref, zt_ref, idx_ref, wts_ref):
    xcol = gp_ref[:, 0:1]
    ycol = gp_ref[:, 1:2]
    zx = zt_ref[0:1, :]
    zy = zt_ref[1:2, :]
    dx = xcol - zx
    dy = ycol - zy
    d2 = dx * dx + dy * dy
    colsf = lax.broadcasted_iota(jnp.int32, (RB2, NZ), 1).astype(jnp.float32)
    m = jnp.min(d2, axis=1, keepdims=True)
    sels = []
    ws = []
    for k in range(KI):
        eq = d2 == m
        sel = jnp.min(jnp.where(eq, colsf, _BIGF), axis=1, keepdims=True)
        sels.append(sel.astype(jnp.int32))
        ws.append(1.0 / jnp.maximum(m, 1e-16))
        if k < KI - 1:
            d2 = jnp.where(colsf == sel, _INF, d2)
            m = jnp.min(d2, axis=1, keepdims=True)
    winv = 1.0 / (ws[0] + ws[1] + ws[2])
    idx_ref[0] = jnp.concatenate(sels + sels + [sels[0], sels[1]], axis=1)
    wts_ref[0] = jnp.concatenate(ws + [winv, winv, winv, winv, winv], axis=1)


def _mlp_body(gp_ref, fi_ref, bands_ref, w0_ref, b0_ref, w1_ref, b1_ref,
              w2_ref, b2_ref, out_ref):
    x = gp_ref[:, 0:1]
    y = gp_ref[:, 1:2]
    bands = bands_ref[...]
    xw = x * bands
    yw = y * bands
    encp = jnp.concatenate(
        [jnp.sin(xw), jnp.sin(yw), jnp.cos(xw), jnp.cos(yw)], axis=1)
    h = (lax.dot_general(encp, w0_ref[...], (((1,), (1,)), ((), ())),
                         preferred_element_type=jnp.float32)
         + b0_ref[...] + fi_ref[:, 0:256])
    h = jnp.maximum(h, 0.0)
    h = (lax.dot_general(h, w1_ref[...], (((1,), (1,)), ((), ())),
                         preferred_element_type=jnp.float32)
         + b1_ref[...] + fi_ref[:, 256:512])
    h = jnp.maximum(h, 0.0)
    o = jnp.sum(h * w2_ref[...], axis=1, keepdims=True) + b2_ref[0]
    out_ref[...] = o


def _sc_gather_body(table_hbm, idx_hbm, w_hbm, out_hbm,
                    idx_v, w_v, rows_a, rows_b, out_a, out_b,
                    gsem_a, gsem_b, osem_a, osem_b):
    wid = lax.axis_index("s") * 2 + lax.axis_index("c")
    base_pt = wid * PW
    pltpu.sync_copy(idx_hbm.at[pl.ds(wid * PW * KI, PW * KI)], idx_v)
    pltpu.sync_copy(w_hbm.at[pl.ds(wid * PW * 4, PW * 4)], w_v)

    def compute_chunk(c, rbuf, obuf):
        def pt_body(p, carry):
            gp = c * CH + p
            w0v = plsc.load_gather(w_v, [jnp.full((16,), 4 * gp + 0, jnp.int32)])
            w1v = plsc.load_gather(w_v, [jnp.full((16,), 4 * gp + 1, jnp.int32)])
            w2v = plsc.load_gather(w_v, [jnp.full((16,), 4 * gp + 2, jnp.int32)])
            wiv = plsc.load_gather(w_v, [jnp.full((16,), 4 * gp + 3, jnp.int32)])
            for j in range(NMOD // 16):
                sl = pl.ds(16 * j, 16)
                acc = (rbuf[KI * p, sl] * w0v
                       + rbuf[KI * p + 1, sl] * w1v
                       + rbuf[KI * p + 2, sl] * w2v)
                obuf[p, sl] = acc * wiv
            return carry
        lax.fori_loop(0, CH, pt_body, 0)

    def gwait(rbuf, gsem):
        pltpu.make_async_copy(table_hbm.at[pl.ds(0, RPC)], rbuf, gsem).wait()

    def owait(obuf, osem):
        pltpu.make_async_copy(
            obuf, out_hbm.at[pl.ds(base_pt, CH)], osem).wait()

    pltpu.async_copy(table_hbm.at[idx_v.at[pl.ds(0, RPC)]], rows_a, gsem_a)

    def pair_body(t, carry):
        c0 = 2 * t
        c1 = 2 * t + 1
        gwait(rows_a, gsem_a)
        pltpu.async_copy(table_hbm.at[idx_v.at[pl.ds(c1 * RPC, RPC)]],
                         rows_b, gsem_b)

        @pl.when(t > 0)
        def _():
            owait(out_a, osem_a)

        compute_chunk(c0, rows_a, out_a)
        pltpu.async_copy(out_a, out_hbm.at[pl.ds(base_pt + c0 * CH, CH)],
                         osem_a)

        gwait(rows_b, gsem_b)

        @pl.when(t + 1 < NCH // 2)
        def _():
            pltpu.async_copy(table_hbm.at[idx_v.at[pl.ds((c0 + 2) * RPC, RPC)]],
                             rows_a, gsem_a)

        @pl.when(t > 0)
        def _():
            owait(out_b, osem_b)

        compute_chunk(c1, rows_b, out_b)
        pltpu.async_copy(out_b, out_hbm.at[pl.ds(base_pt + c1 * CH, CH)],
                         osem_b)
        return carry

    lax.fori_loop(0, NCH // 2, pair_body, 0)
    owait(out_a, osem_a)
    owait(out_b, osem_b)


def _make_feats_call():
    return pl.pallas_call(
        _feats_body,
        grid=(NB1,),
        in_specs=[
            pl.BlockSpec((RB1, 2), lambda b: (b, 0)),
            pl.BlockSpec((2, NZ), lambda b: (0, 0)),
            pl.BlockSpec((NZ, LD), lambda b: (0, 0)),
            pl.BlockSpec((NMOD, LD), lambda b: (0, 0)),
            pl.BlockSpec((NMOD, LD), lambda b: (0, 0)),
            pl.BlockSpec((1, NMOD), lambda b: (0, 0)),
            pl.BlockSpec((1, NMOD), lambda b: (0, 0)),
        ],
        out_specs=pl.BlockSpec((RB1, NMOD), lambda b: (b, 0)),
        out_shape=jax.ShapeDtypeStruct((NZ, NMOD), jnp.float32),
    )


def _make_knn_call():
    return pl.pallas_call(
        _knn_body,
        grid=(NB2,),
        in_specs=[
            pl.BlockSpec((RB2, 2), lambda b: (b, 0)),
            pl.BlockSpec((2, NZ), lambda b: (0, 0)),
        ],
        out_specs=[
            pl.BlockSpec((1, RB2, 8), lambda b: (b, 0, 0)),
            pl.BlockSpec((1, RB2, 8), lambda b: (b, 0, 0)),
        ],
        out_shape=[
            jax.ShapeDtypeStruct((NB2, RB2, 8), jnp.int32),
            jax.ShapeDtypeStruct((NB2, RB2, 8), jnp.float32),
        ],
    )


def _make_mlp_call():
    return pl.pallas_call(
        _mlp_body,
        grid=(NB4,),
        in_specs=[
            pl.BlockSpec((RB4, 2), lambda b: (b, 0)),
            pl.BlockSpec((RB4, NMOD), lambda b: (b, 0)),
            pl.BlockSpec((1, 32), lambda b: (0, 0)),
            pl.BlockSpec((256, 128), lambda b: (0, 0)),
            pl.BlockSpec((1, 256), lambda b: (0, 0)),
            pl.BlockSpec((256, 256), lambda b: (0, 0)),
            pl.BlockSpec((1, 256), lambda b: (0, 0)),
            pl.BlockSpec((1, 256), lambda b: (0, 0)),
            pl.BlockSpec(memory_space=pltpu.SMEM),
        ],
        out_specs=pl.BlockSpec((RB4, 1), lambda b: (b, 0)),
        out_shape=jax.ShapeDtypeStruct((NGP, 1), jnp.float32),
    )


def _make_sc_gather():
    mesh = plsc.VectorSubcoreMesh(core_axis_name="c", subcore_axis_name="s")
    return functools.partial(
        pl.kernel,
        mesh=mesh,
        out_type=jax.ShapeDtypeStruct((NGP, NMOD), jnp.float32),
        scratch_types=[
            pltpu.VMEM((PW * KI,), jnp.int32),
            pltpu.VMEM((PW * 4,), jnp.float32),
            pltpu.VMEM((RPC, NMOD), jnp.float32),
            pltpu.VMEM((RPC, NMOD), jnp.float32),
            pltpu.VMEM((CH, NMOD), jnp.float32),
            pltpu.VMEM((CH, NMOD), jnp.float32),
            pltpu.SemaphoreType.DMA,
            pltpu.SemaphoreType.DMA,
            pltpu.SemaphoreType.DMA,
            pltpu.SemaphoreType.DMA,
        ],
        compiler_params=pltpu.CompilerParams(needs_layout_passes=False),
    )(_sc_gather_body)


_PERM = np.empty((128,), np.int32)
_f = np.arange(32)
_PERM[_f] = 2 * _f
_PERM[32 + _f] = 2 * _f + 1
_PERM[64 + _f] = 64 + 2 * _f
_PERM[96 + _f] = 64 + 2 * _f + 1


def kernel(z_pos, z_features, graph_pos, z_batch, graph_batch,
           sage_l_w, sage_l_b, sage_r_w, sage_r_b,
           w0, b0, w1, b1, w2, b2):
    f32 = jnp.float32
    zt = z_pos.T

    feats = _make_feats_call()(
        z_pos, zt, z_features, sage_l_w, sage_r_w,
        sage_l_b[None, :], sage_r_b[None, :])

    gpad = jnp.concatenate(
        [graph_pos, jnp.zeros((NGP - NG, 2), f32)], axis=0)
    idx3, wts = _make_knn_call()(gpad, zt)
    idx_flat = idx3[:, :, :KI].reshape(-1)
    w_flat = wts[:, :, :4].reshape(-1)

    fi = _make_sc_gather()(feats, idx_flat, w_flat)

    bands = (2.0 ** jnp.linspace(0.0, 10.0, 32)).astype(f32)[None, :]
    w0p = w0[:, _PERM]
    out = _make_mlp_call()(
        gpad, fi, bands, w0p, b0[None, :], w1, b1[None, :], w2, b2)
    return out[:NG]


# speedup vs baseline: 12.0654x; 1.1310x over previous
import functools

import jax
import jax.numpy as jnp
import numpy as np
from jax import lax
from jax.experimental import pallas as pl
from jax.experimental.pallas import tpu as pltpu
from jax.experimental.pallas import tpu_sc as plsc

NZ = 4096
NG = 20000
NGP = 20480
LD = 16
NMOD = 512
KG = 8
KI = 3
RB1 = 512
NB1 = NZ // RB1
RB2 = 512
NB2 = NGP // RB2
RB4 = 512
NB4 = NGP // RB4

SC_WORKERS = 32
PW = NGP // SC_WORKERS
CH = 16
NCH = PW // CH
RPC = CH * KI

_BIGF = np.float32(1e9)
_INF = np.float32(np.inf)


def _feats_body(zp_ref, zt_ref, zf_ref, lw_ref, rw_ref, lb_ref, rb_ref,
                out_ref):
    b = pl.program_id(0)
    xcol = zp_ref[:, 0:1]
    ycol = zp_ref[:, 1:2]
    zx = zt_ref[0:1, :]
    zy = zt_ref[1:2, :]
    dx = xcol - zx
    dy = ycol - zy
    d2 = dx * dx + dy * dy
    rowsf = (lax.broadcasted_iota(jnp.int32, (RB1, NZ), 0).astype(jnp.float32)
             + (b * RB1).astype(jnp.float32))
    colsf = lax.broadcasted_iota(jnp.int32, (RB1, NZ), 1).astype(jnp.float32)
    diag = rowsf == colsf
    d2 = jnp.where(diag, _INF, d2)
    m = jnp.min(d2, axis=1, keepdims=True)
    for k in range(KG):
        eq = d2 == m
        sel = jnp.min(jnp.where(eq, colsf, _BIGF), axis=1, keepdims=True)
        d2 = jnp.where(colsf == sel, _INF, d2)
        if k < KG - 1:
            m = jnp.min(d2, axis=1, keepdims=True)
    wsel = jnp.where(jnp.isinf(d2) & (~diag), 1.0, 0.0).astype(jnp.float32)
    agg = lax.dot_general(wsel, zf_ref[...], (((1,), (0,)), ((), ())),
                          preferred_element_type=jnp.float32) * 0.125
    zfb = zf_ref[pl.ds(b * RB1, RB1), :]
    f = (lax.dot_general(agg, lw_ref[...], (((1,), (1,)), ((), ())),
                         preferred_element_type=jnp.float32)
         + lb_ref[...]
         + lax.dot_general(zfb, rw_ref[...], (((1,), (1,)), ((), ())),
                           preferred_element_type=jnp.float32)
         + rb_ref[...])
    out_ref[...] = f


def _knn_body(gp_ref, zt_ref, idx_ref, wts_ref):
    xcol = gp_ref[:, 0:1]
    ycol = gp_ref[:, 1:2]
    zx = zt_ref[0:1, :]
    zy = zt_ref[1:2, :]
    dx = xcol - zx
    dy = ycol - zy
    d2 = dx * dx + dy * dy
    colsf = lax.broadcasted_iota(jnp.int32, (RB2, NZ), 1).astype(jnp.float32)
    m = jnp.min(d2, axis=1, keepdims=True)
    sels = []
    ws = []
    for k in range(KI):
        eq = d2 == m
        sel = jnp.min(jnp.where(eq, colsf, _BIGF), axis=1, keepdims=True)
        sels.append(sel.astype(jnp.int32))
        ws.append(1.0 / jnp.maximum(m, 1e-16))
        if k < KI - 1:
            d2 = jnp.where(colsf == sel, _INF, d2)
            m = jnp.min(d2, axis=1, keepdims=True)
    winv = 1.0 / (ws[0] + ws[1] + ws[2])
    wn = [w * winv for w in ws]
    idx_ref[0] = jnp.concatenate(sels + sels + [sels[0], sels[1]], axis=1)
    wts_ref[0] = jnp.concatenate(wn + wn + [wn[0], wn[1]], axis=1)


def _sincos(v):
    t = v * np.float32(0.6366197723675814)
    k = jnp.floor(t + np.float32(0.5))
    ki = k.astype(jnp.int32)
    r = v - k * np.float32(1.5703125)
    r = r - k * np.float32(0.0004838705)
    r = r - k * np.float32(-4.371139e-08)
    x2 = r * r
    p = np.float32(-1.9515295891e-4)
    p = p * x2 + np.float32(8.3321608736e-3)
    p = p * x2 + np.float32(-1.6666654611e-1)
    sp = p * x2 * r + r
    q2 = np.float32(2.443315711809948e-5)
    q2 = q2 * x2 + np.float32(-1.388731625493765e-3)
    q2 = q2 * x2 + np.float32(4.166664568298827e-2)
    cp = q2 * x2 * x2 - np.float32(0.5) * x2 + np.float32(1.0)
    q = ki & 3
    odd = (q & 1) == 1
    sinr = jnp.where(odd, cp, sp)
    cosr = jnp.where(odd, sp, cp)
    sv = jnp.where((q & 2) == 2, -sinr, sinr)
    cv = jnp.where(((q + 1) & 2) == 2, -cosr, cosr)
    return sv, cv


def _mlp_body(gp_ref, fi_ref, bands_ref, w0_ref, b0_ref, w1_ref, b1_ref,
              w2_ref, b2_ref, out_ref):
    x = gp_ref[:, 0:1]
    y = gp_ref[:, 1:2]
    bands = bands_ref[...]
    xw = x * bands
    yw = y * bands
    sx, cx = _sincos(xw)
    sy, cy = _sincos(yw)
    encp = jnp.concatenate([sx, sy, cx, cy], axis=1)
    h = (lax.dot_general(encp, w0_ref[...], (((1,), (1,)), ((), ())),
                         preferred_element_type=jnp.float32)
         + b0_ref[...] + fi_ref[:, 0:256])
    h = jnp.maximum(h, 0.0)
    h = (lax.dot_general(h, w1_ref[...], (((1,), (1,)), ((), ())),
                         preferred_element_type=jnp.float32)
         + b1_ref[...] + fi_ref[:, 256:512])
    h = jnp.maximum(h, 0.0)
    o = jnp.sum(h * w2_ref[...], axis=1, keepdims=True) + b2_ref[0]
    out_ref[...] = o


def _sc_gather_body(pw, nch, table_hbm, idx_hbm, w_hbm, out_hbm,
                    idx_v, w_v, rows_0, rows_1, rows_2, rows_3,
                    out_a, out_b,
                    gsem_0, gsem_1, gsem_2, gsem_3, osem_a, osem_b):
    wid = lax.axis_index("s") * 2 + lax.axis_index("c")
    base_pt = wid * pw
    pltpu.sync_copy(idx_hbm.at[pl.ds(base_pt * KI, pw * KI)], idx_v)
    pltpu.sync_copy(w_hbm.at[pl.ds(base_pt * 4, pw * 4)], w_v)

    rows = [rows_0, rows_1, rows_2, rows_3]
    gsems = [gsem_0, gsem_1, gsem_2, gsem_3]
    outs = [out_a, out_b]
    osems = [osem_a, osem_b]

    def gather_chunk(c, lane):
        pltpu.async_copy(table_hbm.at[idx_v.at[pl.ds(c * RPC, RPC)]],
                         rows[lane], gsems[lane])

    def compute_chunk(c, rbuf, obuf):
        @plsc.parallel_loop(0, CH, unroll=2)
        def _(p):
            gp = c * CH + p
            w0v = plsc.load_gather(w_v, [jnp.full((16,), 4 * gp + 0, jnp.int32)])
            w1v = plsc.load_gather(w_v, [jnp.full((16,), 4 * gp + 1, jnp.int32)])
            w2v = plsc.load_gather(w_v, [jnp.full((16,), 4 * gp + 2, jnp.int32)])
            for j in range(NMOD // 16):
                sl = pl.ds(16 * j, 16)
                obuf[p, sl] = (rbuf[KI * p, sl] * w0v
                               + rbuf[KI * p + 1, sl] * w1v
                               + rbuf[KI * p + 2, sl] * w2v)

    def gwait(lane):
        pltpu.make_async_copy(table_hbm.at[pl.ds(0, RPC)], rows[lane],
                              gsems[lane]).wait()

    def owait(ob):
        pltpu.make_async_copy(
            outs[ob], out_hbm.at[pl.ds(base_pt, CH)], osems[ob]).wait()

    gather_chunk(0, 0)
    gather_chunk(1, 1)
    gather_chunk(2, 2)

    def quad_body(t, carry):
        c0 = 4 * t
        for lane in range(4):
            c = c0 + lane
            gwait(lane)

            @pl.when(c + 3 < nch)
            def _():
                gather_chunk(c + 3, (lane + 3) % 4)

            ob = lane % 2

            @pl.when(c >= 2)
            def _():
                owait(ob)

            compute_chunk(c, rows[lane], outs[ob])
            pltpu.async_copy(outs[ob],
                             out_hbm.at[pl.ds(base_pt + c * CH, CH)],
                             osems[ob])
        return carry

    lax.fori_loop(0, nch // 4, quad_body, 0)
    owait(0)
    owait(1)


def _make_feats_call():
    return pl.pallas_call(
        _feats_body,
        grid=(NB1,),
        in_specs=[
            pl.BlockSpec((RB1, 2), lambda b: (b, 0)),
            pl.BlockSpec((2, NZ), lambda b: (0, 0)),
            pl.BlockSpec((NZ, LD), lambda b: (0, 0)),
            pl.BlockSpec((NMOD, LD), lambda b: (0, 0)),
            pl.BlockSpec((NMOD, LD), lambda b: (0, 0)),
            pl.BlockSpec((1, NMOD), lambda b: (0, 0)),
            pl.BlockSpec((1, NMOD), lambda b: (0, 0)),
        ],
        out_specs=pl.BlockSpec((RB1, NMOD), lambda b: (b, 0)),
        out_shape=jax.ShapeDtypeStruct((NZ, NMOD), jnp.float32),
    )


def _make_knn_call():
    return pl.pallas_call(
        _knn_body,
        grid=(NB2,),
        in_specs=[
            pl.BlockSpec((RB2, 2), lambda b: (b, 0)),
            pl.BlockSpec((2, NZ), lambda b: (0, 0)),
        ],
        out_specs=[
            pl.BlockSpec((1, RB2, 8), lambda b: (b, 0, 0)),
            pl.BlockSpec((1, RB2, 8), lambda b: (b, 0, 0)),
        ],
        out_shape=[
            jax.ShapeDtypeStruct((NB2, RB2, 8), jnp.int32),
            jax.ShapeDtypeStruct((NB2, RB2, 8), jnp.float32),
        ],
    )


def _make_mlp_call(npts):
    return pl.pallas_call(
        _mlp_body,
        grid=(npts // RB4,),
        in_specs=[
            pl.BlockSpec((RB4, 2), lambda b: (b, 0)),
            pl.BlockSpec((RB4, NMOD), lambda b: (b, 0)),
            pl.BlockSpec((1, 32), lambda b: (0, 0)),
            pl.BlockSpec((256, 128), lambda b: (0, 0)),
            pl.BlockSpec((1, 256), lambda b: (0, 0)),
            pl.BlockSpec((256, 256), lambda b: (0, 0)),
            pl.BlockSpec((1, 256), lambda b: (0, 0)),
            pl.BlockSpec((1, 256), lambda b: (0, 0)),
            pl.BlockSpec(memory_space=pltpu.SMEM),
        ],
        out_specs=pl.BlockSpec((RB4, 1), lambda b: (b, 0)),
        out_shape=jax.ShapeDtypeStruct((npts, 1), jnp.float32),
    )


def _make_sc_gather(npts):
    pw = npts // SC_WORKERS
    nch = pw // CH
    mesh = plsc.VectorSubcoreMesh(core_axis_name="c", subcore_axis_name="s")
    return functools.partial(
        pl.kernel,
        mesh=mesh,
        out_type=jax.ShapeDtypeStruct((npts, NMOD), jnp.float32),
        scratch_types=[
            pltpu.VMEM((pw * KI,), jnp.int32),
            pltpu.VMEM((pw * 4,), jnp.float32),
            pltpu.VMEM((RPC, NMOD), jnp.float32),
            pltpu.VMEM((RPC, NMOD), jnp.float32),
            pltpu.VMEM((RPC, NMOD), jnp.float32),
            pltpu.VMEM((RPC, NMOD), jnp.float32),
            pltpu.VMEM((CH, NMOD), jnp.float32),
            pltpu.VMEM((CH, NMOD), jnp.float32),
            pltpu.SemaphoreType.DMA,
            pltpu.SemaphoreType.DMA,
            pltpu.SemaphoreType.DMA,
            pltpu.SemaphoreType.DMA,
            pltpu.SemaphoreType.DMA,
            pltpu.SemaphoreType.DMA,
        ],
        compiler_params=pltpu.CompilerParams(needs_layout_passes=False),
    )(functools.partial(_sc_gather_body, pw, nch))


_PERM = np.empty((128,), np.int32)
_f = np.arange(32)
_PERM[_f] = 2 * _f
_PERM[32 + _f] = 2 * _f + 1
_PERM[64 + _f] = 64 + 2 * _f
_PERM[96 + _f] = 64 + 2 * _f + 1


def kernel(z_pos, z_features, graph_pos, z_batch, graph_batch,
           sage_l_w, sage_l_b, sage_r_w, sage_r_b,
           w0, b0, w1, b1, w2, b2):
    f32 = jnp.float32
    zt = z_pos.T

    feats = _make_feats_call()(
        z_pos, zt, z_features, sage_l_w, sage_r_w,
        sage_l_b[None, :], sage_r_b[None, :])

    gpad = jnp.concatenate(
        [graph_pos, jnp.zeros((NGP - NG, 2), f32)], axis=0)
    idx3, wts = _make_knn_call()(gpad, zt)
    idx_flat = idx3[:, :, :KI].reshape(-1)
    w_flat = wts[:, :, :4].reshape(-1)

    half = NGP // 2
    fi0 = _make_sc_gather(half)(
        feats, idx_flat[:half * KI], w_flat[:half * 4])
    fi1 = _make_sc_gather(half)(
        feats, idx_flat[half * KI:], w_flat[half * 4:])

    bands = (2.0 ** jnp.linspace(0.0, 10.0, 32)).astype(f32)[None, :]
    w0p = w0[:, _PERM]
    mlp = _make_mlp_call(half)
    out0 = mlp(gpad[:half], fi0, bands, w0p, b0[None, :],
               w1, b1[None, :], w2, b2)
    out1 = mlp(gpad[half:], fi1, bands, w0p, b0[None, :],
               w1, b1[None, :], w2, b2)
    return jnp.concatenate([out0, out1], axis=0)[:NG]
